# baseline (speedup 1.0000x reference)
import functools

import jax
import jax.numpy as jnp
from jax import lax
from jax.experimental import pallas as pl
from jax.experimental.pallas import tpu as pltpu
from jax.experimental.pallas import tpu_sc as plsc

B, K, D = 4096, 8, 64
NR = 16
NC, NS, L = 2, 16, 16
NW = NC * NS
NB = B // NW
BLK = 256


CB = 2
NCH = NB // CB
RING = 8


OFF_U = 0
OFF_E0 = OFF_U + B
OFF_E1 = OFF_E0 + B
OFF_R1 = OFF_E1 + B * K
OFF_E2 = OFF_R1 + B * K
OFF_R2 = OFF_E2 + B * K * K


def _sc_gather_agg(idx_cat, user_table, entity_table, rel_t):
    mesh = plsc.VectorSubcoreMesh(core_axis_name="c", subcore_axis_name="s")

    @functools.partial(
        pl.kernel,
        mesh=mesh,
        compiler_params=pltpu.CompilerParams(
            needs_layout_passes=False, use_tc_tiling_on_sc=False),
        out_type=(
            jax.ShapeDtypeStruct((B * K * D // 128, 128), jnp.float32),
            jax.ShapeDtypeStruct((B, D), jnp.float32),
            jax.ShapeDtypeStruct((B, NR), jnp.float32),
            jax.ShapeDtypeStruct((B, D), jnp.float32),
        ),
        scratch_types=[
            pltpu.VMEM((NB,), jnp.int32),
            pltpu.VMEM((NB,), jnp.int32),
            pltpu.VMEM((NB * K,), jnp.int32),
            pltpu.VMEM((NB * K * K,), jnp.int32),
            pltpu.VMEM((NB * K + L,), jnp.int32),
            pltpu.VMEM((NB * K * K,), jnp.int32),
            pltpu.VMEM((NB, D), jnp.float32),
            pltpu.VMEM((NB, D), jnp.float32),
            pltpu.VMEM((RING, CB * K, D), jnp.float32),
            pltpu.VMEM((RING, CB * K * K, D), jnp.float32),
            pltpu.VMEM((RING, CB * K * D // 128, 128), jnp.float32),
            pltpu.VMEM((D, NR), jnp.float32),
            pltpu.VMEM((NR,), jnp.float32),
            pltpu.VMEM((NB, D), jnp.float32),
            pltpu.VMEM((NB, NR), jnp.float32),
        ] + [pltpu.SemaphoreType.DMA] * RING,
    )
    def k(idx_h, utab_h, etab_h, relt_h,
          x1_o, d0_o, w1_o, u_o,
          uidx, e0idx, e1idx, e2idx, r1idx, r2idx,
          urows, e0rows, e1rows, e2rows, x1out, relT, exbuf,
          d0s, w1s, *sems):
        wid = lax.axis_index("s") * NC + lax.axis_index("c")
        base = pl.multiple_of(wid * NB, NB)

        pltpu.sync_copy(idx_h.at[pl.ds(OFF_U + base, NB)], uidx)
        pltpu.sync_copy(idx_h.at[pl.ds(OFF_E0 + base, NB)], e0idx)
        pltpu.sync_copy(idx_h.at[pl.ds(OFF_E1 + base * K, NB * K)], e1idx)
        pltpu.sync_copy(idx_h.at[pl.ds(OFF_E2 + base * K * K, NB * K * K)],
                        e2idx)
        pltpu.sync_copy(idx_h.at[pl.ds(OFF_R1 + base * K, NB * K)],
                        r1idx.at[pl.ds(0, NB * K)])
        pltpu.sync_copy(idx_h.at[pl.ds(OFF_R2 + base * K * K, NB * K * K)],
                        r2idx)
        pltpu.sync_copy(relt_h, relT)
        r1idx[pl.ds(NB * K, L)] = jnp.zeros((L,), jnp.int32)

        pltpu.async_copy(utab_h.at[uidx], urows, sems[0]).wait()
        pltpu.async_copy(etab_h.at[e0idx], e0rows, sems[0]).wait()

        lane = lax.iota(jnp.int32, L)
        lo_mask = lane < K
        perm = (lane % 4) * 2 + (lane // 4) % 2

        def fire(c, s):
            o2 = pl.multiple_of(c * (CB * K * K), CB * K * K)
            o1 = pl.multiple_of(c * (CB * K), CB * K)
            pltpu.async_copy(etab_h.at[e2idx.at[pl.ds(o2, CB * K * K)]],
                             e2rows.at[s], sems[s])
            pltpu.async_copy(etab_h.at[e1idx.at[pl.ds(o1, CB * K)]],
                             e1rows.at[s], sems[s])

        def drain(c, s):
            pltpu.make_async_copy(
                etab_h.at[e2idx.at[pl.ds(0, CB * K * K)]],
                e2rows.at[s], sems[s]).wait()
            pltpu.make_async_copy(
                etab_h.at[e1idx.at[pl.ds(0, CB * K)]],
                e1rows.at[s], sems[s]).wait()

            @pl.when(c >= RING)
            def _():
                pltpu.make_async_copy(
                    x1out.at[s], x1_o.at[pl.ds(base * 4, CB * K * D // 128)],
                    sems[s]).wait()

        for s in range(RING - 1):
            fire(s, s)

        def body(j, carry):
            for s in range(RING):
                c = RING * j + s
                drain(c, s)

                nxt = c + (RING - 1)

                @pl.when(nxt < NCH)
                def _():
                    fire(nxt, (s + RING - 1) % RING)

                for boff in range(CB):
                    i = c * CB + boff
                    i64 = pl.multiple_of(i * (K * K), K * K)
                    i8 = pl.multiple_of(i * K, K)
                    eoff2 = boff * K * K
                    eoff1 = boff * K
                    uv = [urows[i, pl.ds(q * L, L)] for q in range(4)]
                    acc = [jnp.zeros((NR,), jnp.float32) for _ in range(4)]
                    for dd in range(D):
                        acc[dd % 4] = (
                            acc[dd % 4] + uv[dd // L][dd % L] * relT[dd, :])
                    s16 = (acc[0] + acc[1]) + (acc[2] + acc[3])
                    exbuf[...] = jnp.exp(s16)

                    wg = []
                    for g in range(4):
                        idxg = r2idx[pl.ds(i64 + g * L, L)]
                        wg.append(plsc.load_gather(exbuf, [idxg]))

                    r1v = plsc.load_gather(r1idx, [i8 + perm])
                    w1g = jnp.where(
                        lo_mask, plsc.load_gather(exbuf, [r1v]), 0.0)
                    s1 = ((w1g[0] + w1g[1]) + (w1g[2] + w1g[3])) + (
                        (w1g[4] + w1g[5]) + (w1g[6] + w1g[7]))
                    w1n = w1g / jnp.full((L,), s1)
                    w1s[i, :] = w1n

                    for n in range(K):
                        va = [jnp.zeros((L,), jnp.float32) for _ in range(4)]
                        ssum = jnp.float32(0.0)
                        for kk in range(K):
                            w = wg[n // 2][(n % 2) * K + kk]
                            ssum = ssum + w
                            row = eoff2 + n * K + kk
                            for q in range(4):
                                va[q] = (va[q]
                                         + w * e2rows[s, row,
                                                      pl.ds(q * L, L)])
                        sv = jnp.full((L,), ssum)
                        prow = (boff * K + n) // 2
                        pcol = (n % 2) * D
                        for q in range(4):
                            x1out[s, prow, pl.ds(pcol + q * L, L)] = (
                                e1rows[s, eoff1 + n, pl.ds(q * L, L)]
                                + va[q] / sv)

                    vb = [jnp.zeros((L,), jnp.float32) for _ in range(4)]
                    for kk in range(K):
                        w = w1n[kk // 2 + (kk % 2) * 4]
                        for q in range(4):
                            vb[q] = (vb[q]
                                     + w * e1rows[s, eoff1 + kk,
                                                  pl.ds(q * L, L)])
                    for q in range(4):
                        d0s[i, pl.ds(q * L, L)] = (
                            e0rows[i, pl.ds(q * L, L)] + vb[q])

                pltpu.async_copy(
                    x1out.at[s],
                    x1_o.at[pl.ds(base * 4 + c * (CB * K * D // 128),
                                  CB * K * D // 128)],
                    sems[s])
            return carry

        lax.fori_loop(0, NCH // RING, body, 0)

        for s in range(RING):
            pltpu.make_async_copy(
                x1out.at[s], x1_o.at[pl.ds(base * 4, CB * K * D // 128)],
                sems[s]).wait()

        pltpu.sync_copy(d0s, d0_o.at[pl.ds(base, NB)])
        pltpu.sync_copy(w1s, w1_o.at[pl.ds(base, NB)])
        pltpu.sync_copy(urows, u_o.at[pl.ds(base, NB)])

    return k(idx_cat, user_table, entity_table, rel_t)


def _tc_body(x1_ref, d0_ref, w1_ref, u_ref, w_ref, b_ref, o_ref):
    wm = w_ref[...]
    bias = b_ref[...]
    x1p = x1_ref[...]
    xa = x1p[:, :D]
    xb = x1p[:, D:]
    h1a = jax.nn.sigmoid(
        jnp.dot(xa, wm, preferred_element_type=jnp.float32) + bias)
    h1b = jax.nn.sigmoid(
        jnp.dot(xb, wm, preferred_element_type=jnp.float32) + bias)
    h1a = h1a.reshape(BLK, K // 2, D)
    h1b = h1b.reshape(BLK, K // 2, D)
    h0 = jax.nn.sigmoid(
        jnp.dot(d0_ref[...], wm, preferred_element_type=jnp.float32) + bias)
    w1e = w1_ref[...][:, :K // 2]
    w1o = w1_ref[...][:, K // 2:K]
    mix = h0 + (jnp.sum(w1e[:, :, None] * h1a, axis=1)
                + jnp.sum(w1o[:, :, None] * h1b, axis=1))
    item = jnp.tanh(
        jnp.dot(mix, wm, preferred_element_type=jnp.float32) + bias)
    o_ref[...] = jax.nn.sigmoid(jnp.sum(u_ref[...] * item, axis=-1))


def _tc_dense(x1, d0, w1p, u, w, bias2d):
    grid = (B // BLK,)
    return pl.pallas_call(
        _tc_body,
        grid=grid,
        in_specs=[
            pl.BlockSpec((BLK * K * D // 128, 128), lambda i: (i, 0)),
            pl.BlockSpec((BLK, D), lambda i: (i, 0)),
            pl.BlockSpec((BLK, NR), lambda i: (i, 0)),
            pl.BlockSpec((BLK, D), lambda i: (i, 0)),
            pl.BlockSpec((D, D), lambda i: (0, 0)),
            pl.BlockSpec((1, D), lambda i: (0, 0)),
        ],
        out_specs=pl.BlockSpec((BLK,), lambda i: (i,)),
        out_shape=jax.ShapeDtypeStruct((B,), jnp.float32),
    )(x1, d0, w1p, u, w, bias2d)


def kernel(users, entities0, entities1, entities2, relations1, relations2,
           user_table, entity_table, rel_table, W, bias):
    idx_cat = jnp.concatenate([
        users.reshape(B).astype(jnp.int32),
        entities0.reshape(B).astype(jnp.int32),
        entities1.reshape(B * K).astype(jnp.int32),
        relations1.reshape(B * K).astype(jnp.int32),
        entities2.reshape(B * K * K).astype(jnp.int32),
        relations2.reshape(B * K * K).astype(jnp.int32),
    ])
    rel_t = rel_table.T
    x1, d0, w1p, u = _sc_gather_agg(idx_cat, user_table, entity_table, rel_t)
    return _tc_dense(x1, d0, w1p, u, W, bias.reshape(1, D))


# speedup vs baseline: 1.0752x; 1.0752x over previous
import functools

import jax
import jax.numpy as jnp
from jax import lax
from jax.experimental import pallas as pl
from jax.experimental.pallas import tpu as pltpu
from jax.experimental.pallas import tpu_sc as plsc

B, K, D = 4096, 8, 64
NR = 16
NC, NS, L = 2, 16, 16
NW = NC * NS
NB = B // NW
BLK = 256


CB = 2
NCH = NB // CB
RING = 4


OFF_U = 0
OFF_E0 = OFF_U + B
OFF_E1 = OFF_E0 + B
OFF_R1 = OFF_E1 + B * K
OFF_E2 = OFF_R1 + B * K
OFF_R2 = OFF_E2 + B * K * K


def _sc_gather_agg(idx_cat, user_table, entity_table, rel_t):
    mesh = plsc.VectorSubcoreMesh(core_axis_name="c", subcore_axis_name="s")

    @functools.partial(
        pl.kernel,
        mesh=mesh,
        compiler_params=pltpu.CompilerParams(
            needs_layout_passes=False, use_tc_tiling_on_sc=False),
        out_type=(
            jax.ShapeDtypeStruct((B * K * D // 128, 128), jnp.float32),
            jax.ShapeDtypeStruct((B, D), jnp.float32),
            jax.ShapeDtypeStruct((B, NR), jnp.float32),
            jax.ShapeDtypeStruct((B, D), jnp.float32),
        ),
        scratch_types=[
            pltpu.VMEM((NB,), jnp.int32),
            pltpu.VMEM((NB,), jnp.int32),
            pltpu.VMEM((NB * K,), jnp.int32),
            pltpu.VMEM((NB * K * K,), jnp.int32),
            pltpu.VMEM((NB * K + L,), jnp.int32),
            pltpu.VMEM((NB * K * K,), jnp.int32),
            pltpu.VMEM((NB, D), jnp.float32),
            pltpu.VMEM((NB, D), jnp.float32),
            pltpu.VMEM((RING, CB * K, D), jnp.float32),
            pltpu.VMEM((RING, CB * K * K, D), jnp.float32),
            pltpu.VMEM((RING, CB * K * D // 128, 128), jnp.float32),
            pltpu.VMEM((D, NR), jnp.float32),
            pltpu.VMEM((NR,), jnp.float32),
            pltpu.VMEM((NB, D), jnp.float32),
            pltpu.VMEM((NB, NR), jnp.float32),
        ] + [pltpu.SemaphoreType.DMA] * RING,
    )
    def k(idx_h, utab_h, etab_h, relt_h,
          x1_o, d0_o, w1_o, u_o,
          uidx, e0idx, e1idx, e2idx, r1idx, r2idx,
          urows, e0rows, e1rows, e2rows, x1out, relT, exbuf,
          d0s, w1s, *sems):
        wid = lax.axis_index("s") * NC + lax.axis_index("c")
        base = pl.multiple_of(wid * NB, NB)

        pltpu.sync_copy(idx_h.at[pl.ds(OFF_U + base, NB)], uidx)
        pltpu.sync_copy(idx_h.at[pl.ds(OFF_E0 + base, NB)], e0idx)
        pltpu.sync_copy(idx_h.at[pl.ds(OFF_E1 + base * K, NB * K)], e1idx)
        pltpu.sync_copy(idx_h.at[pl.ds(OFF_E2 + base * K * K, NB * K * K)],
                        e2idx)
        pltpu.sync_copy(idx_h.at[pl.ds(OFF_R1 + base * K, NB * K)],
                        r1idx.at[pl.ds(0, NB * K)])
        pltpu.sync_copy(idx_h.at[pl.ds(OFF_R2 + base * K * K, NB * K * K)],
                        r2idx)
        pltpu.sync_copy(relt_h, relT)
        r1idx[pl.ds(NB * K, L)] = jnp.zeros((L,), jnp.int32)

        pltpu.async_copy(utab_h.at[uidx], urows, sems[0]).wait()
        pltpu.async_copy(etab_h.at[e0idx], e0rows, sems[0]).wait()

        lane = lax.iota(jnp.int32, L)
        lo_mask = lane < K
        perm = (lane % 4) * 2 + (lane // 4) % 2

        def fire(c, s):
            o2 = pl.multiple_of(c * (CB * K * K), CB * K * K)
            o1 = pl.multiple_of(c * (CB * K), CB * K)
            pltpu.async_copy(etab_h.at[e2idx.at[pl.ds(o2, CB * K * K)]],
                             e2rows.at[s], sems[s])
            pltpu.async_copy(etab_h.at[e1idx.at[pl.ds(o1, CB * K)]],
                             e1rows.at[s], sems[s])

        def drain(c, s):
            pltpu.make_async_copy(
                etab_h.at[e2idx.at[pl.ds(0, CB * K * K)]],
                e2rows.at[s], sems[s]).wait()
            pltpu.make_async_copy(
                etab_h.at[e1idx.at[pl.ds(0, CB * K)]],
                e1rows.at[s], sems[s]).wait()

            @pl.when(c >= RING)
            def _():
                pltpu.make_async_copy(
                    x1out.at[s], x1_o.at[pl.ds(base * 4, CB * K * D // 128)],
                    sems[s]).wait()

        for s in range(RING - 1):
            fire(s, s)

        def body(j, carry):
            for s in range(RING):
                c = RING * j + s
                drain(c, s)

                nxt = c + (RING - 1)

                @pl.when(nxt < NCH)
                def _():
                    fire(nxt, (s + RING - 1) % RING)

                for boff in range(CB):
                    i = c * CB + boff
                    i64 = pl.multiple_of(i * (K * K), K * K)
                    i8 = pl.multiple_of(i * K, K)
                    eoff2 = boff * K * K
                    eoff1 = boff * K
                    uv = [urows[i, pl.ds(q * L, L)] for q in range(4)]
                    acc = [jnp.zeros((NR,), jnp.float32) for _ in range(4)]
                    for dd in range(D):
                        acc[dd % 4] = (
                            acc[dd % 4] + uv[dd // L][dd % L] * relT[dd, :])
                    s16 = (acc[0] + acc[1]) + (acc[2] + acc[3])
                    exbuf[...] = jnp.exp(s16)

                    wg = []
                    for g in range(4):
                        idxg = r2idx[pl.ds(i64 + g * L, L)]
                        wg.append(plsc.load_gather(exbuf, [idxg]))

                    r1v = plsc.load_gather(r1idx, [i8 + perm])
                    w1g = jnp.where(
                        lo_mask, plsc.load_gather(exbuf, [r1v]), 0.0)
                    s1 = ((w1g[0] + w1g[1]) + (w1g[2] + w1g[3])) + (
                        (w1g[4] + w1g[5]) + (w1g[6] + w1g[7]))
                    w1n = w1g / jnp.full((L,), s1)
                    w1s[i, :] = w1n

                    for n in range(K):
                        va = [jnp.zeros((L,), jnp.float32) for _ in range(4)]
                        ssum = jnp.float32(0.0)
                        for kk in range(K):
                            w = wg[n // 2][(n % 2) * K + kk]
                            ssum = ssum + w
                            row = eoff2 + n * K + kk
                            for q in range(4):
                                va[q] = (va[q]
                                         + w * e2rows[s, row,
                                                      pl.ds(q * L, L)])
                        sv = jnp.full((L,), ssum)
                        prow = (boff * K + n) // 2
                        pcol = (n % 2) * D
                        for q in range(4):
                            x1out[s, prow, pl.ds(pcol + q * L, L)] = (
                                e1rows[s, eoff1 + n, pl.ds(q * L, L)]
                                + va[q] / sv)

                    vb = [jnp.zeros((L,), jnp.float32) for _ in range(4)]
                    for kk in range(K):
                        w = w1n[kk // 2 + (kk % 2) * 4]
                        for q in range(4):
                            vb[q] = (vb[q]
                                     + w * e1rows[s, eoff1 + kk,
                                                  pl.ds(q * L, L)])
                    for q in range(4):
                        d0s[i, pl.ds(q * L, L)] = (
                            e0rows[i, pl.ds(q * L, L)] + vb[q])

                pltpu.async_copy(
                    x1out.at[s],
                    x1_o.at[pl.ds(base * 4 + c * (CB * K * D // 128),
                                  CB * K * D // 128)],
                    sems[s])
            return carry

        lax.fori_loop(0, NCH // RING, body, 0)

        for s in range(RING):
            pltpu.make_async_copy(
                x1out.at[s], x1_o.at[pl.ds(base * 4, CB * K * D // 128)],
                sems[s]).wait()

        pltpu.sync_copy(d0s, d0_o.at[pl.ds(base, NB)])
        pltpu.sync_copy(w1s, w1_o.at[pl.ds(base, NB)])
        pltpu.sync_copy(urows, u_o.at[pl.ds(base, NB)])

    return k(idx_cat, user_table, entity_table, rel_t)


def _tc_body(x1_ref, d0_ref, w1_ref, u_ref, w_ref, b_ref, o_ref):
    wm = w_ref[...]
    bias = b_ref[...]
    x1p = x1_ref[...]
    xa = x1p[:, :D]
    xb = x1p[:, D:]
    h1a = jax.nn.sigmoid(
        jnp.dot(xa, wm, preferred_element_type=jnp.float32) + bias)
    h1b = jax.nn.sigmoid(
        jnp.dot(xb, wm, preferred_element_type=jnp.float32) + bias)
    h1a = h1a.reshape(BLK, K // 2, D)
    h1b = h1b.reshape(BLK, K // 2, D)
    h0 = jax.nn.sigmoid(
        jnp.dot(d0_ref[...], wm, preferred_element_type=jnp.float32) + bias)
    w1e = w1_ref[...][:, :K // 2]
    w1o = w1_ref[...][:, K // 2:K]
    mix = h0 + (jnp.sum(w1e[:, :, None] * h1a, axis=1)
                + jnp.sum(w1o[:, :, None] * h1b, axis=1))
    item = jnp.tanh(
        jnp.dot(mix, wm, preferred_element_type=jnp.float32) + bias)
    o_ref[...] = jax.nn.sigmoid(jnp.sum(u_ref[...] * item, axis=-1))


def _tc_dense(x1, d0, w1p, u, w, bias2d):
    grid = (B // BLK,)
    return pl.pallas_call(
        _tc_body,
        grid=grid,
        in_specs=[
            pl.BlockSpec((BLK * K * D // 128, 128), lambda i: (i, 0)),
            pl.BlockSpec((BLK, D), lambda i: (i, 0)),
            pl.BlockSpec((BLK, NR), lambda i: (i, 0)),
            pl.BlockSpec((BLK, D), lambda i: (i, 0)),
            pl.BlockSpec((D, D), lambda i: (0, 0)),
            pl.BlockSpec((1, D), lambda i: (0, 0)),
        ],
        out_specs=pl.BlockSpec((BLK,), lambda i: (i,)),
        out_shape=jax.ShapeDtypeStruct((B,), jnp.float32),
    )(x1, d0, w1p, u, w, bias2d)


def kernel(users, entities0, entities1, entities2, relations1, relations2,
           user_table, entity_table, rel_table, W, bias):
    idx_cat = jnp.concatenate([
        users.reshape(B).astype(jnp.int32),
        entities0.reshape(B).astype(jnp.int32),
        entities1.reshape(B * K).astype(jnp.int32),
        relations1.reshape(B * K).astype(jnp.int32),
        entities2.reshape(B * K * K).astype(jnp.int32),
        relations2.reshape(B * K * K).astype(jnp.int32),
    ])
    rel_t = rel_table.T
    x1, d0, w1p, u = _sc_gather_agg(idx_cat, user_table, entity_table, rel_t)
    return _tc_dense(x1, d0, w1p, u, W, bias.reshape(1, D))


# speedup vs baseline: 1.1035x; 1.0263x over previous
import functools

import jax
import jax.numpy as jnp
from jax import lax
from jax.experimental import pallas as pl
from jax.experimental.pallas import tpu as pltpu
from jax.experimental.pallas import tpu_sc as plsc

B, K, D = 4096, 8, 64
NR = 16
NC, NS, L = 2, 16, 16
NW = NC * NS
NB = B // NW
BLK = 512


CB = 2
NCH = NB // CB
RING = 4


OFF_U = 0
OFF_E0 = OFF_U + B
OFF_E1 = OFF_E0 + B
OFF_R1 = OFF_E1 + B * K
OFF_E2 = OFF_R1 + B * K
OFF_R2 = OFF_E2 + B * K * K


def _sc_gather_agg(idx_cat, user_table, entity_table, rel_t):
    mesh = plsc.VectorSubcoreMesh(core_axis_name="c", subcore_axis_name="s")

    @functools.partial(
        pl.kernel,
        mesh=mesh,
        compiler_params=pltpu.CompilerParams(
            needs_layout_passes=False, use_tc_tiling_on_sc=False),
        out_type=(
            jax.ShapeDtypeStruct((B * K * D // 128, 128), jnp.float32),
            jax.ShapeDtypeStruct((B, D), jnp.float32),
            jax.ShapeDtypeStruct((B, NR), jnp.float32),
            jax.ShapeDtypeStruct((B, D), jnp.float32),
        ),
        scratch_types=[
            pltpu.VMEM((NB,), jnp.int32),
            pltpu.VMEM((NB,), jnp.int32),
            pltpu.VMEM((NB * K,), jnp.int32),
            pltpu.VMEM((NB * K * K,), jnp.int32),
            pltpu.VMEM((NB * K + L,), jnp.int32),
            pltpu.VMEM((NB * K * K,), jnp.int32),
            pltpu.VMEM((NB, D), jnp.float32),
            pltpu.VMEM((NB, D), jnp.float32),
            pltpu.VMEM((RING, CB * K, D), jnp.float32),
            pltpu.VMEM((RING, CB * K * K, D), jnp.float32),
            pltpu.VMEM((RING, CB * K * D // 128, 128), jnp.float32),
            pltpu.VMEM((D, NR), jnp.float32),
            pltpu.VMEM((NR,), jnp.float32),
            pltpu.VMEM((NB, D), jnp.float32),
            pltpu.VMEM((NB, NR), jnp.float32),
        ] + [pltpu.SemaphoreType.DMA] * RING,
    )
    def k(idx_h, utab_h, etab_h, relt_h,
          x1_o, d0_o, w1_o, u_o,
          uidx, e0idx, e1idx, e2idx, r1idx, r2idx,
          urows, e0rows, e1rows, e2rows, x1out, relT, exbuf,
          d0s, w1s, *sems):
        wid = lax.axis_index("s") * NC + lax.axis_index("c")
        base = pl.multiple_of(wid * NB, NB)

        pltpu.sync_copy(idx_h.at[pl.ds(OFF_U + base, NB)], uidx)
        pltpu.sync_copy(idx_h.at[pl.ds(OFF_E0 + base, NB)], e0idx)
        pltpu.sync_copy(idx_h.at[pl.ds(OFF_E1 + base * K, NB * K)], e1idx)
        pltpu.sync_copy(idx_h.at[pl.ds(OFF_E2 + base * K * K, NB * K * K)],
                        e2idx)
        pltpu.sync_copy(idx_h.at[pl.ds(OFF_R1 + base * K, NB * K)],
                        r1idx.at[pl.ds(0, NB * K)])
        pltpu.sync_copy(idx_h.at[pl.ds(OFF_R2 + base * K * K, NB * K * K)],
                        r2idx)
        pltpu.sync_copy(relt_h, relT)
        r1idx[pl.ds(NB * K, L)] = jnp.zeros((L,), jnp.int32)

        cu = pltpu.async_copy(utab_h.at[uidx], urows, sems[0])
        ce = pltpu.async_copy(etab_h.at[e0idx], e0rows, sems[1])
        cu.wait()
        ce.wait()

        lane = lax.iota(jnp.int32, L)
        lo_mask = lane < K
        perm = (lane % 4) * 2 + (lane // 4) % 2

        def fire(c, s):
            o2 = pl.multiple_of(c * (CB * K * K), CB * K * K)
            o1 = pl.multiple_of(c * (CB * K), CB * K)
            pltpu.async_copy(etab_h.at[e2idx.at[pl.ds(o2, CB * K * K)]],
                             e2rows.at[s], sems[s])
            pltpu.async_copy(etab_h.at[e1idx.at[pl.ds(o1, CB * K)]],
                             e1rows.at[s], sems[s])

        def drain(c, s):
            pltpu.make_async_copy(
                etab_h.at[e2idx.at[pl.ds(0, CB * K * K)]],
                e2rows.at[s], sems[s]).wait()
            pltpu.make_async_copy(
                etab_h.at[e1idx.at[pl.ds(0, CB * K)]],
                e1rows.at[s], sems[s]).wait()

            @pl.when(c >= RING)
            def _():
                pltpu.make_async_copy(
                    x1out.at[s], x1_o.at[pl.ds(base * 4, CB * K * D // 128)],
                    sems[s]).wait()

        for s in range(RING - 1):
            fire(s, s)

        def body(j, carry):
            for s in range(RING):
                c = RING * j + s
                drain(c, s)

                nxt = c + (RING - 1)

                @pl.when(nxt < NCH)
                def _():
                    fire(nxt, (s + RING - 1) % RING)

                for boff in range(CB):
                    i = c * CB + boff
                    i64 = pl.multiple_of(i * (K * K), K * K)
                    i8 = pl.multiple_of(i * K, K)
                    eoff2 = boff * K * K
                    eoff1 = boff * K
                    uv = [urows[i, pl.ds(q * L, L)] for q in range(4)]
                    acc = [jnp.zeros((NR,), jnp.float32) for _ in range(4)]
                    for dd in range(D):
                        acc[dd % 4] = (
                            acc[dd % 4] + uv[dd // L][dd % L] * relT[dd, :])
                    s16 = (acc[0] + acc[1]) + (acc[2] + acc[3])
                    exbuf[...] = jnp.exp(s16)

                    wg = []
                    for g in range(4):
                        idxg = r2idx[pl.ds(i64 + g * L, L)]
                        wg.append(plsc.load_gather(exbuf, [idxg]))

                    r1v = plsc.load_gather(r1idx, [i8 + perm])
                    w1g = jnp.where(
                        lo_mask, plsc.load_gather(exbuf, [r1v]), 0.0)
                    s1 = ((w1g[0] + w1g[1]) + (w1g[2] + w1g[3])) + (
                        (w1g[4] + w1g[5]) + (w1g[6] + w1g[7]))
                    w1n = w1g / jnp.full((L,), s1)
                    w1s[i, :] = w1n

                    for n in range(K):
                        va = [jnp.zeros((L,), jnp.float32) for _ in range(4)]
                        ssum = jnp.float32(0.0)
                        for kk in range(K):
                            w = wg[n // 2][(n % 2) * K + kk]
                            ssum = ssum + w
                            row = eoff2 + n * K + kk
                            for q in range(4):
                                va[q] = (va[q]
                                         + w * e2rows[s, row,
                                                      pl.ds(q * L, L)])
                        sv = jnp.full((L,), ssum)
                        prow = (boff * K + n) // 2
                        pcol = (n % 2) * D
                        for q in range(4):
                            x1out[s, prow, pl.ds(pcol + q * L, L)] = (
                                e1rows[s, eoff1 + n, pl.ds(q * L, L)]
                                + va[q] / sv)

                    vb = [jnp.zeros((L,), jnp.float32) for _ in range(4)]
                    for kk in range(K):
                        w = w1n[kk // 2 + (kk % 2) * 4]
                        for q in range(4):
                            vb[q] = (vb[q]
                                     + w * e1rows[s, eoff1 + kk,
                                                  pl.ds(q * L, L)])
                    for q in range(4):
                        d0s[i, pl.ds(q * L, L)] = (
                            e0rows[i, pl.ds(q * L, L)] + vb[q])

                pltpu.async_copy(
                    x1out.at[s],
                    x1_o.at[pl.ds(base * 4 + c * (CB * K * D // 128),
                                  CB * K * D // 128)],
                    sems[s])
            return carry

        lax.fori_loop(0, NCH // RING, body, 0)

        for s in range(RING):
            pltpu.make_async_copy(
                x1out.at[s], x1_o.at[pl.ds(base * 4, CB * K * D // 128)],
                sems[s]).wait()

        pltpu.sync_copy(d0s, d0_o.at[pl.ds(base, NB)])
        pltpu.sync_copy(w1s, w1_o.at[pl.ds(base, NB)])
        pltpu.sync_copy(urows, u_o.at[pl.ds(base, NB)])

    return k(idx_cat, user_table, entity_table, rel_t)


def _tc_body(x1_ref, d0_ref, w1_ref, u_ref, w_ref, b_ref, o_ref):
    wm = w_ref[...]
    bias = b_ref[...]
    x1p = x1_ref[...]
    xa = x1p[:, :D]
    xb = x1p[:, D:]
    h1a = jax.nn.sigmoid(
        jnp.dot(xa, wm, preferred_element_type=jnp.float32) + bias)
    h1b = jax.nn.sigmoid(
        jnp.dot(xb, wm, preferred_element_type=jnp.float32) + bias)
    h1a = h1a.reshape(BLK, K // 2, D)
    h1b = h1b.reshape(BLK, K // 2, D)
    h0 = jax.nn.sigmoid(
        jnp.dot(d0_ref[...], wm, preferred_element_type=jnp.float32) + bias)
    w1e = w1_ref[...][:, :K // 2]
    w1o = w1_ref[...][:, K // 2:K]
    mix = h0 + (jnp.sum(w1e[:, :, None] * h1a, axis=1)
                + jnp.sum(w1o[:, :, None] * h1b, axis=1))
    item = jnp.tanh(
        jnp.dot(mix, wm, preferred_element_type=jnp.float32) + bias)
    o_ref[...] = jax.nn.sigmoid(jnp.sum(u_ref[...] * item, axis=-1))


def _tc_dense(x1, d0, w1p, u, w, bias2d):
    grid = (B // BLK,)
    return pl.pallas_call(
        _tc_body,
        grid=grid,
        in_specs=[
            pl.BlockSpec((BLK * K * D // 128, 128), lambda i: (i, 0)),
            pl.BlockSpec((BLK, D), lambda i: (i, 0)),
            pl.BlockSpec((BLK, NR), lambda i: (i, 0)),
            pl.BlockSpec((BLK, D), lambda i: (i, 0)),
            pl.BlockSpec((D, D), lambda i: (0, 0)),
            pl.BlockSpec((1, D), lambda i: (0, 0)),
        ],
        out_specs=pl.BlockSpec((BLK,), lambda i: (i,)),
        out_shape=jax.ShapeDtypeStruct((B,), jnp.float32),
    )(x1, d0, w1p, u, w, bias2d)


def kernel(users, entities0, entities1, entities2, relations1, relations2,
           user_table, entity_table, rel_table, W, bias):
    idx_cat = jnp.concatenate([
        users.reshape(B).astype(jnp.int32),
        entities0.reshape(B).astype(jnp.int32),
        entities1.reshape(B * K).astype(jnp.int32),
        relations1.reshape(B * K).astype(jnp.int32),
        entities2.reshape(B * K * K).astype(jnp.int32),
        relations2.reshape(B * K * K).astype(jnp.int32),
    ])
    rel_t = rel_table.T
    x1, d0, w1p, u = _sc_gather_agg(idx_cat, user_table, entity_table, rel_t)
    return _tc_dense(x1, d0, w1p, u, W, bias.reshape(1, D))
